# baseline (speedup 1.0000x reference)
import functools

import jax
import jax.numpy as jnp
from jax import lax
from jax.experimental import pallas as pl
from jax.experimental.pallas import tpu as pltpu
from jax.experimental.pallas import tpu_sc as plsc

D = 64
N_WORKERS = 32
SUB = 128
LANES = 16


def _neg_norm(x):
    xs = jnp.maximum(x, jnp.float32(1e-30))
    i = lax.bitcast_convert_type(xs, jnp.int32)
    y = lax.bitcast_convert_type(jnp.int32(0x5F3759DF) - (i >> 1), jnp.float32)
    half = jnp.float32(0.5) * xs
    for _ in range(3):
        y = y * (jnp.float32(1.5) - half * y * y)
    return -(xs * y)


def kernel(heads, rels, tails, times, entity_table, relation_table, time_table):
    B = heads.shape[0]
    rows_per_w = B // N_WORKERS
    n_sub = rows_per_w // SUB

    mesh = plsc.VectorSubcoreMesh(core_axis_name="c", subcore_axis_name="s")

    @functools.partial(
        pl.kernel,
        mesh=mesh,
        compiler_params=pltpu.CompilerParams(
            needs_layout_passes=False, use_tc_tiling_on_sc=False
        ),
        out_type=jax.ShapeDtypeStruct((B,), jnp.float32),
        scratch_types=[
            pltpu.VMEM((rows_per_w,), jnp.int32),
            pltpu.VMEM((rows_per_w,), jnp.int32),
            pltpu.VMEM((rows_per_w,), jnp.int32),
            pltpu.VMEM((rows_per_w,), jnp.int32),
            pltpu.VMEM((SUB, D), jnp.float32),
            pltpu.VMEM((SUB, D), jnp.float32),
            pltpu.VMEM((SUB, D), jnp.float32),
            pltpu.VMEM((SUB, D), jnp.float32),
            pltpu.VMEM((rows_per_w,), jnp.float32),
            pltpu.SemaphoreType.DMA,
        ],
    )
    def k(heads_h, rels_h, tails_h, times_h, ent_h, rel_h, time_h, out_h,
          hidx, ridx, tidx, midx, hb, rb, tb, mb, ob, sem):
        wid = lax.axis_index("s") * 2 + lax.axis_index("c")
        base = wid * rows_per_w

        pltpu.sync_copy(heads_h.at[pl.ds(base, rows_per_w)], hidx)
        pltpu.sync_copy(rels_h.at[pl.ds(base, rows_per_w)], ridx)
        pltpu.sync_copy(tails_h.at[pl.ds(base, rows_per_w)], tidx)
        pltpu.sync_copy(times_h.at[pl.ds(base, rows_per_w)], midx)

        lane = lax.iota(jnp.int32, LANES)

        for j in range(n_sub):
            sl = pl.ds(j * SUB, SUB)
            c1 = pltpu.async_copy(ent_h.at[hidx.at[sl]], hb, sem)
            c2 = pltpu.async_copy(rel_h.at[ridx.at[sl]], rb, sem)
            c3 = pltpu.async_copy(ent_h.at[tidx.at[sl]], tb, sem)
            c4 = pltpu.async_copy(time_h.at[midx.at[sl]], mb, sem)
            c1.wait()
            c2.wait()
            c3.wait()
            c4.wait()

            def gbody(g, _):
                v = jnp.zeros((LANES,), jnp.float32)
                for r in range(LANES):
                    row = g * LANES + r
                    part = jnp.zeros((LANES,), jnp.float32)
                    for c in range(D // LANES):
                        cs = pl.ds(c * LANES, LANES)
                        s = hb[row, cs] + rb[row, cs] + mb[row, cs] - tb[row, cs]
                        part = part + s * s
                    v = jnp.where(lane == jnp.int32(r), jnp.sum(part), v)
                ob[pl.ds(j * SUB + g * LANES, LANES)] = _neg_norm(v)
                return _

            lax.fori_loop(0, SUB // LANES, gbody, jnp.int32(0))

        pltpu.sync_copy(ob, out_h.at[pl.ds(base, rows_per_w)])

    return k(heads, rels, tails, times, entity_table, relation_table, time_table)
